# baseline (speedup 1.0000x reference)
import functools

import jax
import jax.numpy as jnp
from jax import lax
from jax.experimental import pallas as pl
from jax.experimental.pallas import tpu as pltpu
from jax.experimental.pallas import tpu_sc as plsc

NUM_EMB = 1000000
DIM = 32
BATCH = 16384
HIST = 20
TOTAL = BATCH * HIST

_info = plsc.get_sparse_core_info()
_NC = _info.num_cores
_NS = _info.num_subcores
_NW = _NC * _NS

_TCOLS = 8192
_TGRID = -(-NUM_EMB // _TCOLS)


def _wt_body(wt_ref, out_ref):
    out_ref[...] = wt_ref[...].T


_w_transpose = pl.pallas_call(
    _wt_body,
    grid=(_TGRID,),
    in_specs=[pl.BlockSpec((DIM, _TCOLS), lambda j: (0, j))],
    out_specs=pl.BlockSpec((_TCOLS, DIM), lambda j: (j, 0)),
    out_shape=jax.ShapeDtypeStruct((NUM_EMB, DIM), jnp.float32),
)

_B_PER_W = TOTAL // _NW
_CHUNK = 1024
_NCHUNKS = _B_PER_W // _CHUNK
_NBUF = 3


def _body(idx_hbm, table_hbm, out_hbm, idx_v, rows_v, *sems):
    g_sems = sems[:_NBUF]
    s_sems = sems[_NBUF:]
    wid = lax.axis_index("s") * _NC + lax.axis_index("c")
    base = wid * _B_PER_W
    pltpu.sync_copy(idx_hbm.at[pl.ds(base, _B_PER_W)], idx_v)

    def start_gather(c):
        b = c % _NBUF
        idx_sl = idx_v.at[pl.ds(c * _CHUNK, _CHUNK)]
        return pltpu.async_copy(table_hbm.at[idx_sl], rows_v.at[b], g_sems[b])

    gathers = [start_gather(c) for c in range(min(_NBUF, _NCHUNKS))]
    gathers += [None] * (_NCHUNKS - len(gathers))
    scatters = [None] * _NCHUNKS
    for c in range(_NCHUNKS):
        b = c % _NBUF
        gathers[c].wait()
        scatters[c] = pltpu.async_copy(
            rows_v.at[b], out_hbm.at[pl.ds(base + c * _CHUNK, _CHUNK)], s_sems[b]
        )
        nc = c + _NBUF
        if nc < _NCHUNKS:
            scatters[c].wait()
            gathers[nc] = start_gather(nc)
    for c in range(max(0, _NCHUNKS - _NBUF), _NCHUNKS):
        scatters[c].wait()


_gather = pl.kernel(
    _body,
    out_type=jax.ShapeDtypeStruct((TOTAL, DIM), jnp.float32),
    mesh=plsc.VectorSubcoreMesh(core_axis_name="c", subcore_axis_name="s"),
    scratch_types=[
        pltpu.VMEM((_B_PER_W,), jnp.int32),
        pltpu.VMEM((_NBUF, _CHUNK, DIM), jnp.float32),
    ]
    + [pltpu.SemaphoreType.DMA] * (2 * _NBUF),
    compiler_params=pltpu.CompilerParams(use_tc_tiling_on_sc=False),
)

_BBLK = 2048
_BGRID = BATCH // _BBLK


def _ot_body(rows_ref, out_ref):
    out_ref[0] = rows_ref[0].T


_out_transpose = pl.pallas_call(
    _ot_body,
    grid=(HIST, _BGRID),
    in_specs=[pl.BlockSpec((1, _BBLK, DIM), lambda h, j: (h, j, 0))],
    out_specs=pl.BlockSpec((1, DIM, _BBLK), lambda h, j: (h, 0, j)),
    out_shape=jax.ShapeDtypeStruct((HIST, DIM, BATCH), jnp.float32),
)


@jax.jit
def kernel(token_ids, weights):
    idx = jnp.reshape(token_ids.T, (TOTAL,)).astype(jnp.int32)
    w_lin = _w_transpose(weights.T)
    rows = _gather(idx, w_lin)
    out_t = _out_transpose(jnp.reshape(rows, (HIST, BATCH, DIM)))
    return jnp.transpose(out_t, (2, 0, 1))


# speedup vs baseline: 1.4248x; 1.4248x over previous
import functools

import jax
import jax.numpy as jnp
from jax import lax
from jax.experimental import pallas as pl
from jax.experimental.pallas import tpu as pltpu
from jax.experimental.pallas import tpu_sc as plsc

NUM_EMB = 1000000
DIM = 32
BATCH = 16384
HIST = 20
TOTAL = BATCH * HIST

_info = plsc.get_sparse_core_info()
_NC = _info.num_cores
_NS = _info.num_subcores
_NW = _NC * _NS
_L = _info.num_lanes

_B_PER_W = TOTAL // _NW
_CHUNK = 512
_NCHUNKS = _B_PER_W // _CHUNK
_GBUF = 3
_TBUF = 2
_TSTR = _CHUNK + 1


def _body(idx_hbm, table_hbm, out_hbm, idx_v, rows_v, t_v, *sems):
    g_sems = sems[:_GBUF]
    o_sems = sems[_GBUF:]
    wid = lax.axis_index("s") * _NC + lax.axis_index("c")
    base = wid * _B_PER_W
    pltpu.sync_copy(idx_hbm.at[pl.ds(base, _B_PER_W)], idx_v)

    iota = lax.iota(jnp.int32, _L)
    d_lo = iota
    d_hi = iota + _L

    def start_gather(c):
        g = c % _GBUF
        idx_sl = idx_v.at[pl.ds(c * _CHUNK, _CHUNK)]
        return pltpu.async_copy(table_hbm.at[idx_sl], rows_v.at[g], g_sems[g])

    def start_out(c):
        t = c % _TBUF
        j0 = base + c * _CHUNK
        h = j0 // BATCH
        b0 = j0 % BATCH
        return pltpu.async_copy(
            t_v.at[t, :, pl.ds(0, _CHUNK)],
            out_hbm.at[h, :, pl.ds(b0, _CHUNK)],
            o_sems[t],
        )

    gathers = [start_gather(c) for c in range(min(_GBUF, _NCHUNKS))]
    gathers += [None] * (_NCHUNKS - len(gathers))
    outs = [None] * _NCHUNKS
    for c in range(_NCHUNKS):
        g = c % _GBUF
        t = c % _TBUF
        gathers[c].wait()
        if c >= _TBUF:
            outs[c - _TBUF].wait()

        def transpose_one(l, _, g=g, t=t):
            x0 = rows_v[g, l, pl.ds(0, _L)]
            x1 = rows_v[g, l, pl.ds(_L, _L)]
            lv = jnp.broadcast_to(l, (_L,))
            tv = jnp.broadcast_to(t, (_L,))
            plsc.store_scatter(t_v, [tv, d_lo, lv], x0)
            plsc.store_scatter(t_v, [tv, d_hi, lv], x1)
            return _

        lax.fori_loop(0, _CHUNK, transpose_one, 0, unroll=8)
        outs[c] = start_out(c)
        nc = c + _GBUF
        if nc < _NCHUNKS:
            gathers[nc] = start_gather(nc)
    for c in range(_NCHUNKS - _TBUF, _NCHUNKS):
        outs[c].wait()


_gather = pl.kernel(
    _body,
    out_type=jax.ShapeDtypeStruct((HIST, DIM, BATCH), jnp.float32),
    mesh=plsc.VectorSubcoreMesh(core_axis_name="c", subcore_axis_name="s"),
    scratch_types=[
        pltpu.VMEM((_B_PER_W,), jnp.int32),
        pltpu.VMEM((_GBUF, _CHUNK, DIM), jnp.float32),
        pltpu.VMEM((_TBUF, DIM, _TSTR), jnp.float32),
    ]
    + [pltpu.SemaphoreType.DMA] * (_GBUF + _TBUF),
    compiler_params=pltpu.CompilerParams(
        use_tc_tiling_on_sc=False, needs_layout_passes=False
    ),
)


@jax.jit
def kernel(token_ids, weights):
    idx = jnp.reshape(token_ids.T, (TOTAL,)).astype(jnp.int32)
    out_t = _gather(idx, weights)
    return jnp.transpose(out_t, (2, 0, 1))
